# baseline (device time: 29185 ns/iter reference)
import jax
import jax.numpy as jnp
from jax import lax
from jax.experimental import pallas as pl
from jax.experimental.pallas import tpu as pltpu

CH = 128
MAX_CHUNKS = 8


def kernel(x, dest):
    m, n = x.shape
    my_y = lax.axis_index("y")
    xb = x.astype(jnp.bfloat16)

    order = jnp.argsort(dest == my_y, stable=True)
    sorted_x = xb[order]
    cs = jnp.sum(dest != my_y).astype(jnp.int32)

    def body(cs_ref, x_ref, recv_ref, send_sems, recv_sems):
        my_x = lax.axis_index("x")
        yy = lax.axis_index("y")
        my_z = lax.axis_index("z")
        peer = (my_x, 1 - yy, my_z)

        n_chunks = (cs_ref[0] + CH - 1) // CH

        barrier = pltpu.get_barrier_semaphore()
        pl.semaphore_signal(
            barrier, inc=1, device_id=peer, device_id_type=pl.DeviceIdType.MESH
        )
        pl.semaphore_wait(barrier, 1)

        rdmas = []
        for i in range(MAX_CHUNKS):
            rdma = pltpu.make_async_remote_copy(
                src_ref=x_ref.at[pl.ds(i * CH, CH)],
                dst_ref=recv_ref.at[pl.ds(i * CH, CH)],
                send_sem=send_sems.at[i],
                recv_sem=recv_sems.at[i],
                device_id=peer,
                device_id_type=pl.DeviceIdType.MESH,
            )
            rdmas.append(rdma)

            @pl.when(i < n_chunks)
            def _():
                rdma.start()

        for i in range(MAX_CHUNKS):

            @pl.when(i < n_chunks)
            def _():
                rdmas[i].wait()

    recv = pl.pallas_call(
        body,
        out_shape=jax.ShapeDtypeStruct((m, n), jnp.bfloat16),
        in_specs=[
            pl.BlockSpec(memory_space=pltpu.SMEM),
            pl.BlockSpec(memory_space=pltpu.VMEM),
        ],
        out_specs=pl.BlockSpec(memory_space=pltpu.VMEM),
        scratch_shapes=[
            pltpu.SemaphoreType.DMA((MAX_CHUNKS,)),
            pltpu.SemaphoreType.DMA((MAX_CHUNKS,)),
        ],
        compiler_params=pltpu.CompilerParams(collective_id=0),
    )(cs.reshape(1), sorted_x)

    j = jnp.arange(m)
    is0 = my_y == 0
    use_local = jnp.where(is0, j < m - cs, j >= cs)
    local_idx = jnp.where(is0, cs + j, j)
    recv_idx = jnp.where(is0, j - (m - cs), j)
    idx = jnp.where(use_local, local_idx, m + recv_idx)
    combined = jnp.concatenate([sorted_x, recv], axis=0)
    return combined[idx]


# device time: 19082 ns/iter; 1.5295x vs baseline; 1.5295x over previous
import jax
import jax.numpy as jnp
from jax import lax
from jax.experimental import pallas as pl
from jax.experimental.pallas import tpu as pltpu

CH = 128
MAX_CHUNKS = 8
BIG = 3000


def kernel(x, dest):
    m, n = x.shape
    my_y = lax.axis_index("y")
    is_peer = (dest != my_y).astype(jnp.int32)
    c = jnp.cumsum(is_peer).astype(jnp.int32)
    cs = c[m - 1].reshape(1)
    dest_row = dest.reshape(1, m)
    c_row = c.reshape(1, m)

    def body(cs_ref, dest_ref, c_ref, x_ref, out_ref, sendbuf, recv, send_sems, recv_sems):
        my_x = lax.axis_index("x")
        me = lax.axis_index("y")
        my_z = lax.axis_index("z")
        peer_id = (my_x, 1 - me, my_z)

        csv = cs_ref[0]
        n_chunks = (csv + CH - 1) // CH

        xb = x_ref[...].astype(jnp.bfloat16)
        peer_row = dest_ref[...] != me
        cum = c_ref[...]
        iota_out = lax.broadcasted_iota(jnp.int32, (m, m), 0)

        q_send = jnp.where(peer_row, cum - 1, BIG)
        p_send = (iota_out == q_send).astype(jnp.bfloat16)
        sendbuf[...] = jnp.dot(
            p_send, xb, preferred_element_type=jnp.float32
        ).astype(jnp.bfloat16)

        barrier = pltpu.get_barrier_semaphore()
        pl.semaphore_signal(
            barrier, inc=1, device_id=peer_id, device_id_type=pl.DeviceIdType.MESH
        )
        pl.semaphore_wait(barrier, 1)

        rdmas = []
        for i in range(MAX_CHUNKS):
            rdma = pltpu.make_async_remote_copy(
                src_ref=sendbuf.at[pl.ds(i * CH, CH)],
                dst_ref=recv.at[pl.ds(i * CH, CH)],
                send_sem=send_sems.at[i],
                recv_sem=recv_sems.at[i],
                device_id=peer_id,
                device_id_type=pl.DeviceIdType.MESH,
            )
            rdmas.append(rdma)

            @pl.when(i < n_chunks)
            def _():
                rdma.start()

        r_row = lax.broadcasted_iota(jnp.int32, (1, m), 1)
        kept_rank = r_row - cum
        pos_k = kept_rank + jnp.where(me == 0, 0, csv)
        q_loc = jnp.where(peer_row, BIG, pos_k)
        p_loc = (iota_out == q_loc).astype(jnp.bfloat16)
        out_loc = jnp.dot(
            p_loc, xb, preferred_element_type=jnp.float32
        ).astype(jnp.bfloat16)

        for i in range(MAX_CHUNKS):

            @pl.when(i < n_chunks)
            def _():
                rdmas[i].wait()

        shift = jnp.where(me == 0, m - csv, 0)
        rolled = pltpu.roll(recv[...], shift, axis=0)
        j = lax.broadcasted_iota(jnp.int32, (m, 1), 0)
        lo = jnp.where(me == 0, m - csv, 0)
        hi = jnp.where(me == 0, m, csv)
        mask_recv = (j >= lo) & (j < hi)
        out_ref[...] = jnp.where(mask_recv, rolled, out_loc)

    return pl.pallas_call(
        body,
        out_shape=jax.ShapeDtypeStruct((m, n), jnp.bfloat16),
        in_specs=[
            pl.BlockSpec(memory_space=pltpu.SMEM),
            pl.BlockSpec(memory_space=pltpu.VMEM),
            pl.BlockSpec(memory_space=pltpu.VMEM),
            pl.BlockSpec(memory_space=pltpu.VMEM),
        ],
        out_specs=pl.BlockSpec(memory_space=pltpu.VMEM),
        scratch_shapes=[
            pltpu.VMEM((m, n), jnp.bfloat16),
            pltpu.VMEM((m, n), jnp.bfloat16),
            pltpu.SemaphoreType.DMA((MAX_CHUNKS,)),
            pltpu.SemaphoreType.DMA((MAX_CHUNKS,)),
        ],
        compiler_params=pltpu.CompilerParams(collective_id=0),
    )(cs, dest_row, c_row, x)


# device time: 16311 ns/iter; 1.7893x vs baseline; 1.1699x over previous
import jax
import jax.numpy as jnp
from jax import lax
from jax.experimental import pallas as pl
from jax.experimental.pallas import tpu as pltpu

CH = 128
N_CH = 8
BIG = 3000


def kernel(x, dest):
    m, n = x.shape
    my_y = lax.axis_index("y")
    is_peer = (dest != my_y).astype(jnp.int32)
    c = jnp.cumsum(is_peer).astype(jnp.int32)
    cs = c[m - 1].reshape(1)
    dest_row = dest.reshape(1, m)
    c_row = c.reshape(1, m)

    def body(
        cs_ref, dest_ref, c_ref, x_ref,
        out_ref,
        xv, sendbuf, recv, in_sem, send_sems, recv_sems,
    ):
        my_x = lax.axis_index("x")
        me = lax.axis_index("y")
        my_z = lax.axis_index("z")
        peer_id = (my_x, 1 - me, my_z)

        barrier = pltpu.get_barrier_semaphore()
        pl.semaphore_signal(
            barrier, inc=1, device_id=peer_id, device_id_type=pl.DeviceIdType.MESH
        )

        in_copy = pltpu.make_async_copy(x_ref, xv, in_sem)
        in_copy.start()

        csv = cs_ref[0]
        peer_row = dest_ref[...] != me
        cum = c_ref[...]

        s_out = jnp.where(me == 0, 0, m - csv)
        q_send = jnp.where(peer_row, cum - 1 + s_out, BIG)

        r_row = lax.broadcasted_iota(jnp.int32, (1, m), 1)
        q_loc = jnp.where(peer_row, BIG, r_row - cum + jnp.where(me == 0, 0, csv))

        s_in = jnp.where(me == 0, m - csv, 0)
        k0s = s_out // CH
        k1s = (s_out + csv + CH - 1) // CH
        k0r = s_in // CH
        k1r = (s_in + csv + CH - 1) // CH

        in_copy.wait()
        xb = xv[...].astype(jnp.bfloat16)

        pl.semaphore_wait(barrier, 1)

        rdmas = []
        for i in range(N_CH):
            rdma = pltpu.make_async_remote_copy(
                src_ref=sendbuf.at[pl.ds(i * CH, CH)],
                dst_ref=recv.at[pl.ds(i * CH, CH)],
                send_sem=send_sems.at[i],
                recv_sem=recv_sems.at[i],
                device_id=peer_id,
                device_id_type=pl.DeviceIdType.MESH,
            )
            rdmas.append(rdma)

            @pl.when((i >= k0s) & (i < k1s))
            def _():
                u = lax.broadcasted_iota(jnp.int32, (CH, m), 0) + i * CH
                p_chunk = (u == q_send).astype(jnp.bfloat16)
                sendbuf[pl.ds(i * CH, CH)] = jnp.dot(
                    p_chunk, xb, preferred_element_type=jnp.float32
                ).astype(jnp.bfloat16)
                rdma.start()

        iota_out = lax.broadcasted_iota(jnp.int32, (m, m), 0)
        p_loc = (iota_out == q_loc).astype(jnp.bfloat16)
        out_loc = jnp.dot(p_loc, xb, preferred_element_type=jnp.float32).astype(
            jnp.bfloat16
        )

        j = lax.broadcasted_iota(jnp.int32, (CH, 1), 0)
        for i in range(N_CH):

            @pl.when((i >= k0r) & (i < k1r))
            def _():
                rdmas[i].wait_recv()

            ji = j + i * CH
            mask = (ji >= s_in) & (ji < s_in + csv)
            out_ref[pl.ds(i * CH, CH)] = jnp.where(
                mask,
                recv[pl.ds(i * CH, CH)],
                out_loc[i * CH : (i + 1) * CH],
            )

        for i in range(N_CH):

            @pl.when((i >= k0s) & (i < k1s))
            def _():
                rdmas[i].wait_send()

    return pl.pallas_call(
        body,
        out_shape=jax.ShapeDtypeStruct((m, n), jnp.bfloat16),
        in_specs=[
            pl.BlockSpec(memory_space=pltpu.SMEM),
            pl.BlockSpec(memory_space=pltpu.VMEM),
            pl.BlockSpec(memory_space=pltpu.VMEM),
            pl.BlockSpec(memory_space=pl.ANY),
        ],
        out_specs=pl.BlockSpec(memory_space=pltpu.VMEM),
        scratch_shapes=[
            pltpu.VMEM((m, n), jnp.float32),
            pltpu.VMEM((m, n), jnp.bfloat16),
            pltpu.VMEM((m, n), jnp.bfloat16),
            pltpu.SemaphoreType.DMA,
            pltpu.SemaphoreType.DMA((N_CH,)),
            pltpu.SemaphoreType.DMA((N_CH,)),
        ],
        compiler_params=pltpu.CompilerParams(collective_id=0),
    )(cs, dest_row, c_row, x)


# device time: 14156 ns/iter; 2.0617x vs baseline; 1.1522x over previous
import jax
import jax.numpy as jnp
from jax import lax
from jax.experimental import pallas as pl
from jax.experimental.pallas import tpu as pltpu

CH = 128
N_CH = 8
BIG = 3000


def kernel(x, dest):
    m, n = x.shape
    dest_row = dest.reshape(1, m)

    def body(
        dest_ref, x_ref,
        out_ref,
        xv, sendbuf, recv, in_sem, send_sems, recv_sems,
    ):
        my_x = lax.axis_index("x")
        me = lax.axis_index("y")
        my_z = lax.axis_index("z")
        peer_id = (my_x, 1 - me, my_z)

        barrier = pltpu.get_barrier_semaphore()
        pl.semaphore_signal(
            barrier, inc=1, device_id=peer_id, device_id_type=pl.DeviceIdType.MESH
        )

        in_copy = pltpu.make_async_copy(x_ref, xv, in_sem)
        in_copy.start()

        peer_row = dest_ref[...] != me
        cnt = peer_row.astype(jnp.int32)
        csv = jnp.sum(cnt)
        lane = lax.broadcasted_iota(jnp.int32, (1, m), 1)
        cum = cnt
        k = 1
        while k < m:
            cum = cum + jnp.where(lane >= k, pltpu.roll(cum, k, axis=1), 0)
            k *= 2

        s_out = jnp.where(me == 0, 0, m - csv)
        q_send = jnp.where(peer_row, cum - 1 + s_out, BIG)

        r_row = lax.broadcasted_iota(jnp.int32, (1, m), 1)
        q_loc = jnp.where(peer_row, BIG, r_row - cum + jnp.where(me == 0, 0, csv))

        s_in = jnp.where(me == 0, m - csv, 0)
        k0s = s_out // CH
        k1s = (s_out + csv + CH - 1) // CH
        k0r = s_in // CH
        k1r = (s_in + csv + CH - 1) // CH

        in_copy.wait()
        xb = xv[...].astype(jnp.bfloat16)

        pl.semaphore_wait(barrier, 1)

        rdmas = []
        for i in range(N_CH):
            rdma = pltpu.make_async_remote_copy(
                src_ref=sendbuf.at[pl.ds(i * CH, CH)],
                dst_ref=recv.at[pl.ds(i * CH, CH)],
                send_sem=send_sems.at[i],
                recv_sem=recv_sems.at[i],
                device_id=peer_id,
                device_id_type=pl.DeviceIdType.MESH,
            )
            rdmas.append(rdma)

            @pl.when((i >= k0s) & (i < k1s))
            def _():
                u = lax.broadcasted_iota(jnp.int32, (CH, m), 0) + i * CH
                p_chunk = (u == q_send).astype(jnp.bfloat16)
                sendbuf[pl.ds(i * CH, CH)] = jnp.dot(
                    p_chunk, xb, preferred_element_type=jnp.float32
                ).astype(jnp.bfloat16)
                rdma.start()

        iota_out = lax.broadcasted_iota(jnp.int32, (m, m), 0)
        p_loc = (iota_out == q_loc).astype(jnp.bfloat16)
        out_loc = jnp.dot(p_loc, xb, preferred_element_type=jnp.float32).astype(
            jnp.bfloat16
        )

        j = lax.broadcasted_iota(jnp.int32, (CH, 1), 0)
        for i in range(N_CH):

            @pl.when((i >= k0r) & (i < k1r))
            def _():
                rdmas[i].wait_recv()

            ji = j + i * CH
            mask = (ji >= s_in) & (ji < s_in + csv)
            out_ref[pl.ds(i * CH, CH)] = jnp.where(
                mask,
                recv[pl.ds(i * CH, CH)],
                out_loc[i * CH : (i + 1) * CH],
            )

        for i in range(N_CH):

            @pl.when((i >= k0s) & (i < k1s))
            def _():
                rdmas[i].wait_send()

    return pl.pallas_call(
        body,
        out_shape=jax.ShapeDtypeStruct((m, n), jnp.bfloat16),
        in_specs=[
            pl.BlockSpec(memory_space=pltpu.VMEM),
            pl.BlockSpec(memory_space=pl.ANY),
        ],
        out_specs=pl.BlockSpec(memory_space=pltpu.VMEM),
        scratch_shapes=[
            pltpu.VMEM((m, n), jnp.float32),
            pltpu.VMEM((m, n), jnp.bfloat16),
            pltpu.VMEM((m, n), jnp.bfloat16),
            pltpu.SemaphoreType.DMA,
            pltpu.SemaphoreType.DMA((N_CH,)),
            pltpu.SemaphoreType.DMA((N_CH,)),
        ],
        compiler_params=pltpu.CompilerParams(collective_id=0),
    )(dest_row, x)
